# baseline (speedup 1.0000x reference)
import functools

import jax
import jax.numpy as jnp
from jax import lax
from jax.experimental import pallas as pl
from jax.experimental.pallas import tpu as pltpu
from jax.experimental.pallas import tpu_sc as plsc

_VOCAB = 1000000
_EMBED = 64
_B = 4096
_L = 200
_N = _B * _L

_NC = 2
_NS = 16
_NW = _NC * _NS
_PER_W = _N // _NW
_S = 1
_CH = _S * 128
_STEPS = _PER_W // _CH

_mesh = plsc.VectorSubcoreMesh(core_axis_name="c", subcore_axis_name="s")


@functools.partial(
    pl.kernel,
    mesh=_mesh,
    out_type=jax.ShapeDtypeStruct((_N, _EMBED), jnp.float32),
    scratch_types=[
        pltpu.VMEM((2, _S, 128), jnp.int32),
        pltpu.VMEM((2, _S, 128), jnp.int32),
        pltpu.VMEM((2, _S, 128, 128), jnp.float32),
        pltpu.VMEM((2, _S, 128, _EMBED), jnp.float32),
        pltpu.SemaphoreType.DMA,
        pltpu.SemaphoreType.DMA,
        pltpu.SemaphoreType.DMA,
        pltpu.SemaphoreType.DMA,
    ],
    compiler_params=pltpu.CompilerParams(needs_layout_passes=False),
)
def _emb_lookup(x_hbm, table_hbm, out_hbm, idx2, pidx2, rows2, outv2, g0, g1, s0, s1):
    wid = lax.axis_index("s") * _NC + lax.axis_index("c")
    base = wid * _PER_W
    gsems = (g0, g1)
    ssems = (s0, s1)

    def start_chunk(ci, b):
        off = base + ci * _CH
        for k in range(_S):
            pltpu.sync_copy(x_hbm.at[pl.ds(off + k * 128, 128)], idx2.at[b, k])
            for j in range(8):
                v = idx2[b, k, pl.ds(j * 16, 16)]
                pidx2[b, k, pl.ds(j * 16, 16)] = lax.shift_right_logical(v, 1)
            pltpu.make_async_copy(
                table_hbm.at[pidx2.at[b, k]], rows2.at[b, k], gsems[b]
            ).start()

    def wait_chunk(b):
        for k in range(_S):
            pltpu.make_async_copy(
                table_hbm.at[pidx2.at[b, k]], rows2.at[b, k], gsems[b]
            ).wait()

    def start_store(ci, b):
        off = base + ci * _CH
        for k in range(_S):
            pltpu.make_async_copy(
                outv2.at[b, k],
                out_hbm.at[pl.ds(off + k * 128, 128)],
                ssems[b],
            ).start()

    def wait_store(ci, b):
        off = base + ci * _CH
        for k in range(_S):
            pltpu.make_async_copy(
                outv2.at[b, k],
                out_hbm.at[pl.ds(off + k * 128, 128)],
                ssems[b],
            ).wait()

    lanes = lax.iota(jnp.int32, 16)

    def scale(b):
        def group(g, c):
            i0 = g * 16
            rv = lanes + i0
            for k in range(_S):
                iv = idx2[b, k, pl.ds(i0, 16)]
                ov = lax.shift_left(iv & 1, 6)

                def col(ci_, c_):
                    cv = jnp.full((16,), ci_, jnp.int32)
                    src = plsc.load_gather(rows2.at[b, k], [rv, ov + cv])
                    plsc.store_scatter(outv2.at[b, k], [rv, cv], src * 0.125)
                    return c_

                lax.fori_loop(0, _EMBED, col, 0)
            return c

        lax.fori_loop(0, 8, group, 0)

    start_chunk(0, 0)

    def step(it, carry):
        for b in range(2):
            ci = it * 2 + b
            nb = b ^ 1
            nci = ci + 1

            @pl.when(nci < _STEPS)
            def _prefetch():
                @pl.when(nci >= 2)
                def _drain():
                    wait_store(nci - 2, nb)

                start_chunk(nci, nb)

            wait_chunk(b)
            scale(b)
            start_store(ci, b)
        return carry

    lax.fori_loop(0, _STEPS // 2, step, 0)

    wait_store(_STEPS - 2, 0)
    wait_store(_STEPS - 1, 1)


def kernel(x, table):
    xf = x.reshape(_N)
    t2 = table.reshape(_VOCAB // 2, 2 * _EMBED)
    out = _emb_lookup(xf, t2)
    return out.reshape(_B, _L, _EMBED)


# speedup vs baseline: 1.6077x; 1.6077x over previous
import functools

import jax
import jax.numpy as jnp
from jax import lax
from jax.experimental import pallas as pl
from jax.experimental.pallas import tpu as pltpu
from jax.experimental.pallas import tpu_sc as plsc

_VOCAB = 1000000
_EMBED = 64
_B = 4096
_L = 200
_N = _B * _L

_NC = 2
_NS = 16
_NW = _NC * _NS
_PER_W = _N // _NW
_S = 1
_CH = _S * 128
_STEPS = _PER_W // _CH

_mesh = plsc.VectorSubcoreMesh(core_axis_name="c", subcore_axis_name="s")


@functools.partial(
    pl.kernel,
    mesh=_mesh,
    out_type=jax.ShapeDtypeStruct((_N, _EMBED), jnp.float32),
    scratch_types=[
        pltpu.VMEM((2, _S, 144), jnp.int32),
        pltpu.VMEM((2, _S, 128), jnp.int32),
        pltpu.VMEM((2, _S, 128, 128), jnp.float32),
        pltpu.VMEM((2, _S, 128, _EMBED), jnp.float32),
        pltpu.SemaphoreType.DMA,
        pltpu.SemaphoreType.DMA,
        pltpu.SemaphoreType.DMA,
        pltpu.SemaphoreType.DMA,
    ],
    compiler_params=pltpu.CompilerParams(needs_layout_passes=False),
)
def _emb_lookup(x_hbm, table_hbm, out_hbm, idx2, pidx2, rows2, outv2, g0, g1, s0, s1):
    wid = lax.axis_index("s") * _NC + lax.axis_index("c")
    base = wid * _PER_W
    gsems = (g0, g1)
    ssems = (s0, s1)

    def start_chunk(ci, b):
        off = base + ci * _CH
        for k in range(_S):
            pltpu.sync_copy(x_hbm.at[pl.ds(off + k * 128, 128)], idx2.at[b, k, pl.ds(0, 128)])
            for j in range(8):
                v = idx2[b, k, pl.ds(j * 16, 16)]
                pidx2[b, k, pl.ds(j * 16, 16)] = lax.shift_right_logical(v, 1)
            pltpu.make_async_copy(
                table_hbm.at[pidx2.at[b, k]], rows2.at[b, k], gsems[b]
            ).start()

    def wait_chunk(b):
        for k in range(_S):
            pltpu.make_async_copy(
                table_hbm.at[pidx2.at[b, k]], rows2.at[b, k], gsems[b]
            ).wait()

    def start_store(ci, b):
        off = base + ci * _CH
        for k in range(_S):
            pltpu.make_async_copy(
                outv2.at[b, k],
                out_hbm.at[pl.ds(off + k * 128, 128)],
                ssems[b],
            ).start()

    def wait_store(ci, b):
        off = base + ci * _CH
        for k in range(_S):
            pltpu.make_async_copy(
                outv2.at[b, k],
                out_hbm.at[pl.ds(off + k * 128, 128)],
                ssems[b],
            ).wait()

    def scale(b):
        def body(i, c):
            for k in range(_S):
                iv = idx2[b, k, pl.ds(i, 16)]
                o = lax.shift_left(iv[0] & 1, 6)
                for j in range(_EMBED // 16):
                    sl = rows2[b, k, i, pl.ds(o + j * 16, 16)]
                    outv2[b, k, i, pl.ds(j * 16, 16)] = sl * 0.125
            return c

        lax.fori_loop(0, 128, body, 0)

    start_chunk(0, 0)

    def step(it, carry):
        for b in range(2):
            ci = it * 2 + b
            nb = b ^ 1
            nci = ci + 1

            @pl.when(nci < _STEPS)
            def _prefetch():
                @pl.when(nci >= 2)
                def _drain():
                    wait_store(nci - 2, nb)

                start_chunk(nci, nb)

            wait_chunk(b)
            scale(b)
            start_store(ci, b)
        return carry

    lax.fori_loop(0, _STEPS // 2, step, 0)

    wait_store(_STEPS - 2, 0)
    wait_store(_STEPS - 1, 1)


def kernel(x, table):
    xf = x.reshape(_N)
    t2 = table.reshape(_VOCAB // 2, 2 * _EMBED)
    out = _emb_lookup(xf, t2)
    return out.reshape(_B, _L, _EMBED)


# speedup vs baseline: 2.6287x; 1.6351x over previous
import functools

import jax
import jax.numpy as jnp
from jax import lax
from jax.experimental import pallas as pl
from jax.experimental.pallas import tpu as pltpu
from jax.experimental.pallas import tpu_sc as plsc

_VOCAB = 1000000
_EMBED = 64
_B = 4096
_L = 200
_N = _B * _L

_NC = 2
_NS = 16
_NW = _NC * _NS
_PER_W = _N // _NW
_S = 1
_CH = _S * 128
_STEPS = _PER_W // _CH

_mesh = plsc.VectorSubcoreMesh(core_axis_name="c", subcore_axis_name="s")


@functools.partial(
    pl.kernel,
    mesh=_mesh,
    out_type=jax.ShapeDtypeStruct((_N, _EMBED), jnp.float32),
    scratch_types=[
        pltpu.VMEM((2, _S, 144), jnp.int32),
        pltpu.VMEM((2, _S, 128), jnp.int32),
        pltpu.VMEM((2, _S, 128, 128), jnp.float32),
        pltpu.VMEM((2, _S, 128, _EMBED), jnp.float32),
        pltpu.SemaphoreType.DMA,
        pltpu.SemaphoreType.DMA,
        pltpu.SemaphoreType.DMA,
        pltpu.SemaphoreType.DMA,
    ],
    compiler_params=pltpu.CompilerParams(needs_layout_passes=False),
)
def _emb_lookup(x_hbm, table_hbm, out_hbm, idx2, pidx2, rows2, outv2, g0, g1, s0, s1):
    wid = lax.axis_index("s") * _NC + lax.axis_index("c")
    base = wid * _PER_W
    gsems = (g0, g1)
    ssems = (s0, s1)

    def start_chunk(ci, b):
        off = base + ci * _CH
        for k in range(_S):
            pltpu.sync_copy(x_hbm.at[pl.ds(off + k * 128, 128)], idx2.at[b, k, pl.ds(0, 128)])
            for j in range(8):
                v = idx2[b, k, pl.ds(j * 16, 16)]
                pidx2[b, k, pl.ds(j * 16, 16)] = lax.shift_right_logical(v, 1)
            pltpu.make_async_copy(
                table_hbm.at[pidx2.at[b, k]], rows2.at[b, k], gsems[b]
            ).start()

    def wait_chunk(b):
        for k in range(_S):
            pltpu.make_async_copy(
                table_hbm.at[pidx2.at[b, k]], rows2.at[b, k], gsems[b]
            ).wait()

    def start_store(ci, b):
        off = base + ci * _CH
        for k in range(_S):
            pltpu.make_async_copy(
                outv2.at[b, k],
                out_hbm.at[pl.ds(off + k * 128, 128)],
                ssems[b],
            ).start()

    def wait_store(ci, b):
        off = base + ci * _CH
        for k in range(_S):
            pltpu.make_async_copy(
                outv2.at[b, k],
                out_hbm.at[pl.ds(off + k * 128, 128)],
                ssems[b],
            ).wait()

    def scale(b):
        def body(i, c):
            for k in range(_S):
                for j in range(_EMBED // 16):
                    sl = rows2[b, k, i, pl.ds(j * 16, 16)]
                    outv2[b, k, i, pl.ds(j * 16, 16)] = sl * 0.125
            return c

        lax.fori_loop(0, 128, body, 0)

    start_chunk(0, 0)

    def step(it, carry):
        for b in range(2):
            ci = it * 2 + b
            nb = b ^ 1
            nci = ci + 1

            @pl.when(nci < _STEPS)
            def _prefetch():
                @pl.when(nci >= 2)
                def _drain():
                    wait_store(nci - 2, nb)

                start_chunk(nci, nb)

            wait_chunk(b)
            scale(b)
            start_store(ci, b)
        return carry

    lax.fori_loop(0, _STEPS // 2, step, 0)

    wait_store(_STEPS - 2, 0)
    wait_store(_STEPS - 1, 1)


def kernel(x, table):
    xf = x.reshape(_N)
    t2 = table.reshape(_VOCAB // 2, 2 * _EMBED)
    out = _emb_lookup(xf, t2)
    return out.reshape(_B, _L, _EMBED)


# speedup vs baseline: 2.7897x; 1.0613x over previous
import functools

import jax
import jax.numpy as jnp
from jax import lax
from jax.experimental import pallas as pl
from jax.experimental.pallas import tpu as pltpu
from jax.experimental.pallas import tpu_sc as plsc

_VOCAB = 1000000
_EMBED = 64
_B = 4096
_L = 200
_N = _B * _L

_NC = 2
_NS = 16
_NW = _NC * _NS
_PER_W = _N // _NW
_S = 1
_CH = _S * 128
_STEPS = _PER_W // _CH

_mesh = plsc.VectorSubcoreMesh(core_axis_name="c", subcore_axis_name="s")


@functools.partial(
    pl.kernel,
    mesh=_mesh,
    out_type=jax.ShapeDtypeStruct((_N, _EMBED), jnp.float32),
    scratch_types=[
        pltpu.VMEM((2, _S, 144), jnp.int32),
        pltpu.VMEM((2, _S, 128), jnp.int32),
        pltpu.VMEM((2, _S, 128, 128), jnp.float32),
        pltpu.VMEM((2, _S, 128, _EMBED), jnp.float32),
        pltpu.SemaphoreType.DMA,
        pltpu.SemaphoreType.DMA,
        pltpu.SemaphoreType.DMA,
        pltpu.SemaphoreType.DMA,
    ],
    compiler_params=pltpu.CompilerParams(needs_layout_passes=False),
)
def _emb_lookup(x_hbm, table_hbm, out_hbm, idx2, pidx2, rows2, outv2, g0, g1, s0, s1):
    wid = lax.axis_index("s") * _NC + lax.axis_index("c")
    base = wid * _PER_W
    gsems = (g0, g1)
    ssems = (s0, s1)

    def start_chunk(ci, b):
        off = base + ci * _CH
        for k in range(_S):
            pltpu.sync_copy(x_hbm.at[pl.ds(off + k * 128, 128)], idx2.at[b, k, pl.ds(0, 128)])
            pltpu.make_async_copy(
                table_hbm.at[idx2.at[b, k, pl.ds(0, 128)]], rows2.at[b, k], gsems[b]
            ).start()

    def wait_chunk(b):
        for k in range(_S):
            pltpu.make_async_copy(
                table_hbm.at[idx2.at[b, k, pl.ds(0, 128)]], rows2.at[b, k], gsems[b]
            ).wait()

    def start_store(ci, b):
        off = base + ci * _CH
        for k in range(_S):
            pltpu.make_async_copy(
                outv2.at[b, k],
                out_hbm.at[pl.ds(off + k * 128, 128)],
                ssems[b],
            ).start()

    def wait_store(ci, b):
        off = base + ci * _CH
        for k in range(_S):
            pltpu.make_async_copy(
                outv2.at[b, k],
                out_hbm.at[pl.ds(off + k * 128, 128)],
                ssems[b],
            ).wait()

    def scale(b):
        def body(i, c):
            for k in range(_S):
                for j in range(_EMBED // 16):
                    sl = rows2[b, k, i, pl.ds(j * 16, 16)]
                    outv2[b, k, i, pl.ds(j * 16, 16)] = sl * 0.125
            return c

        lax.fori_loop(0, 128, body, 0)

    start_chunk(0, 0)

    def step(it, carry):
        for b in range(2):
            ci = it * 2 + b
            nb = b ^ 1
            nci = ci + 1

            @pl.when(nci < _STEPS)
            def _prefetch():
                @pl.when(nci >= 2)
                def _drain():
                    wait_store(nci - 2, nb)

                start_chunk(nci, nb)

            wait_chunk(b)
            scale(b)
            start_store(ci, b)
        return carry

    lax.fori_loop(0, _STEPS // 2, step, 0)

    wait_store(_STEPS - 2, 0)
    wait_store(_STEPS - 1, 1)


def kernel(x, table):
    xf = x.reshape(_N)
    t2 = jnp.pad(table, ((0, 0), (0, _EMBED)))
    out = _emb_lookup(xf, t2)
    return out.reshape(_B, _L, _EMBED)


# speedup vs baseline: 2.9238x; 1.0481x over previous
import functools

import jax
import jax.numpy as jnp
from jax import lax
from jax.experimental import pallas as pl
from jax.experimental.pallas import tpu as pltpu
from jax.experimental.pallas import tpu_sc as plsc

_VOCAB = 1000000
_EMBED = 64
_B = 4096
_L = 200
_N = _B * _L

_NC = 2
_NS = 16
_NW = _NC * _NS
_PER_W = _N // _NW
_S = 1
_CH = _S * 128
_STEPS = _PER_W // _CH

_mesh = plsc.VectorSubcoreMesh(core_axis_name="c", subcore_axis_name="s")


@functools.partial(
    pl.kernel,
    mesh=_mesh,
    out_type=jax.ShapeDtypeStruct((_N, _EMBED), jnp.float32),
    scratch_types=[
        pltpu.VMEM((2, _S, 128), jnp.int32),
        pltpu.VMEM((2, _S, 128, 128), jnp.float32),
        pltpu.VMEM((2, _S, 128, _EMBED), jnp.float32),
        pltpu.SemaphoreType.DMA,
        pltpu.SemaphoreType.DMA,
        pltpu.SemaphoreType.DMA,
        pltpu.SemaphoreType.DMA,
        pltpu.SemaphoreType.DMA,
        pltpu.SemaphoreType.DMA,
    ],
    compiler_params=pltpu.CompilerParams(needs_layout_passes=False),
)
def _emb_lookup(x_hbm, table_hbm, out_hbm, idx2, rows2, outv2, g0, g1, s0, s1, i0, i1):
    wid = lax.axis_index("s") * _NC + lax.axis_index("c")
    base = wid * _PER_W
    gsems = (g0, g1)
    ssems = (s0, s1)
    isems = (i0, i1)

    def start_idx(ci, b):
        off = base + ci * _CH
        for k in range(_S):
            pltpu.make_async_copy(
                x_hbm.at[pl.ds(off + k * 128, 128)], idx2.at[b, k], isems[b]
            ).start()

    def wait_idx(ci, b):
        off = base + ci * _CH
        for k in range(_S):
            pltpu.make_async_copy(
                x_hbm.at[pl.ds(off + k * 128, 128)], idx2.at[b, k], isems[b]
            ).wait()

    def start_chunk(ci, b):
        for k in range(_S):
            pltpu.make_async_copy(
                table_hbm.at[idx2.at[b, k]], rows2.at[b, k], gsems[b]
            ).start()

    def wait_chunk(b):
        for k in range(_S):
            pltpu.make_async_copy(
                table_hbm.at[idx2.at[b, k]], rows2.at[b, k], gsems[b]
            ).wait()

    def start_store(ci, b):
        off = base + ci * _CH
        for k in range(_S):
            pltpu.make_async_copy(
                outv2.at[b, k],
                out_hbm.at[pl.ds(off + k * 128, 128)],
                ssems[b],
            ).start()

    def wait_store(ci, b):
        off = base + ci * _CH
        for k in range(_S):
            pltpu.make_async_copy(
                outv2.at[b, k],
                out_hbm.at[pl.ds(off + k * 128, 128)],
                ssems[b],
            ).wait()

    def scale(b):
        def body(i, c):
            for k in range(_S):
                for j in range(_EMBED // 16):
                    sl = rows2[b, k, i, pl.ds(j * 16, 16)]
                    outv2[b, k, i, pl.ds(j * 16, 16)] = sl * 0.125
            return c

        lax.fori_loop(0, 128, body, 0)

    start_idx(0, 0)
    wait_idx(0, 0)
    start_chunk(0, 0)
    start_idx(1, 1)

    def step(it, carry):
        for b in range(2):
            ci = it * 2 + b
            nb = b ^ 1
            nci = ci + 1

            @pl.when(nci < _STEPS)
            def _prefetch():
                @pl.when(nci >= 2)
                def _drain():
                    wait_store(nci - 2, nb)

                wait_idx(nci, nb)
                start_chunk(nci, nb)

            wait_chunk(b)

            @pl.when(ci + 2 < _STEPS)
            def _idx_ahead():
                start_idx(ci + 2, b)

            scale(b)
            start_store(ci, b)
        return carry

    lax.fori_loop(0, _STEPS // 2, step, 0)

    wait_store(_STEPS - 2, 0)
    wait_store(_STEPS - 1, 1)


def kernel(x, table):
    xf = x.reshape(_N)
    t2 = jnp.pad(table, ((0, 0), (0, _EMBED)))
    out = _emb_lookup(xf, t2)
    return out.reshape(_B, _L, _EMBED)
